# baseline (speedup 1.0000x reference)
import functools

import jax
import jax.numpy as jnp
from jax import lax
from jax.experimental import pallas as pl
from jax.experimental.pallas import tpu as pltpu
from jax.experimental.pallas import tpu_sc as plsc

N_NODES = 10000
D = 128

NC = 2
NS = 16
NW = NC * NS

K = 64
NBUF = 3
N_TAB = 10240
ROWS_PER_TILE = N_TAB // NS


def _sc_body(ept, e_tot, x_hbm, ei_hbm, agg_out, deg_out,
             sidx_all, didx_all, rows, ones_v, degz_v, gsems, agg_sh, deg_sh):
    n_full = ept // K
    rem = ept % K
    n_ring = (n_full // NBUF) * NBUF

    c = lax.axis_index("c")
    s = lax.axis_index("s")
    wid = c * NS + s
    base = wid * ept

    zero16 = jnp.zeros((16,), jnp.float32)
    one16 = jnp.ones((16,), jnp.float32)

    pltpu.sync_copy(ei_hbm.at[pl.ds(base, ept)], sidx_all)
    pltpu.sync_copy(ei_hbm.at[pl.ds(e_tot + base, ept)], didx_all)

    def _zrow(i, _):
        for j in range(D // 16):
            rows[0, i, pl.ds(j * 16, 16)] = zero16
        return 0
    lax.fori_loop(0, K, _zrow, 0)

    for j in range(K // 16):
        ones_v[pl.ds(j * 16, 16)] = one16

    def _zdeg(i, _):
        degz_v[pl.ds(i * 16, 16)] = zero16
        return 0
    lax.fori_loop(0, ROWS_PER_TILE // 16, _zdeg, 0)

    for bblk in range(ROWS_PER_TILE // K):
        pltpu.sync_copy(rows.at[0],
                        agg_sh.at[pl.ds(s * ROWS_PER_TILE + bblk * K, K)])
    pltpu.sync_copy(degz_v, deg_sh.at[pl.ds(s * ROWS_PER_TILE, ROWS_PER_TILE)])

    plsc.subcore_barrier()

    for b in range(min(NBUF, n_ring)):
        pltpu.async_copy(x_hbm.at[sidx_all.at[pl.ds(b * K, K)]], rows.at[b],
                         gsems[b])

    def _outer(go, _):
        for b in range(NBUF):
            g = go * NBUF + b
            pltpu.make_async_copy(x_hbm.at[sidx_all.at[pl.ds(g * K, K)]],
                                  rows.at[b], gsems[b]).wait()
            pltpu.sync_copy(rows.at[b],
                            agg_sh.at[didx_all.at[pl.ds(g * K, K)]], add=True)
            pltpu.sync_copy(ones_v,
                            deg_sh.at[didx_all.at[pl.ds(g * K, K)]], add=True)

            @pl.when(g + NBUF < n_ring)
            def _refill():
                pltpu.async_copy(
                    x_hbm.at[sidx_all.at[pl.ds((g + NBUF) * K, K)]],
                    rows.at[b], gsems[b])
        return 0
    lax.fori_loop(0, n_ring // NBUF, _outer, 0)

    for g in range(n_ring, n_full):
        b = g - n_ring
        pltpu.sync_copy(x_hbm.at[sidx_all.at[pl.ds(g * K, K)]], rows.at[b])
        pltpu.sync_copy(rows.at[b],
                        agg_sh.at[didx_all.at[pl.ds(g * K, K)]], add=True)
        pltpu.sync_copy(ones_v,
                        deg_sh.at[didx_all.at[pl.ds(g * K, K)]], add=True)
    if rem:
        off = n_full * K
        pltpu.sync_copy(x_hbm.at[sidx_all.at[pl.ds(off, rem)]],
                        rows.at[0, pl.ds(0, rem)])
        pltpu.sync_copy(rows.at[0, pl.ds(0, rem)],
                        agg_sh.at[didx_all.at[pl.ds(off, rem)]], add=True)
        pltpu.sync_copy(ones_v.at[pl.ds(0, rem)],
                        deg_sh.at[didx_all.at[pl.ds(off, rem)]], add=True)

    plsc.subcore_barrier()

    pltpu.sync_copy(agg_sh.at[pl.ds(s * ROWS_PER_TILE, ROWS_PER_TILE)],
                    agg_out.at[c, pl.ds(s * ROWS_PER_TILE, ROWS_PER_TILE)])
    pltpu.sync_copy(deg_sh.at[pl.ds(s * ROWS_PER_TILE, ROWS_PER_TILE)],
                    deg_out.at[c, pl.ds(s * ROWS_PER_TILE, ROWS_PER_TILE)])


def _segment_mean_sc(x, edge_index, ept):
    e_tot = edge_index.size // 2
    mesh = plsc.VectorSubcoreMesh(core_axis_name="c", subcore_axis_name="s")
    return pl.kernel(
        functools.partial(_sc_body, ept, e_tot),
        out_type=(
            jax.ShapeDtypeStruct((NC, N_TAB, D), jnp.float32),
            jax.ShapeDtypeStruct((NC, N_TAB), jnp.float32),
        ),
        mesh=mesh,
        scratch_types=[
            pltpu.VMEM((ept,), jnp.int32),
            pltpu.VMEM((ept,), jnp.int32),
            pltpu.VMEM((NBUF, K, D), jnp.float32),
            pltpu.VMEM((K,), jnp.float32),
            pltpu.VMEM((ROWS_PER_TILE,), jnp.float32),
            [pltpu.SemaphoreType.DMA] * NBUF,
            pltpu.VMEM_SHARED((N_TAB, D), jnp.float32),
            pltpu.VMEM_SHARED((N_TAB,), jnp.float32),
        ],
        name="gcn_segment_mean_sc",
    )(x, edge_index.reshape(-1))


def _tc_body(agg_ref, deg_ref, w_ref, b_ref, out_ref):
    agg = agg_ref[0] + agg_ref[1]
    deg = deg_ref[0, :, 0] + deg_ref[1, :, 0]
    mean = agg / jnp.maximum(deg, 1.0)[:, None]
    out_ref[...] = (
        jnp.dot(mean, w_ref[...].T, preferred_element_type=jnp.float32)
        + b_ref[...]
    )


def _linear_tc(agg, deg, W, b):
    bs = 1000
    grid = (N_NODES // bs,)
    return pl.pallas_call(
        _tc_body,
        grid=grid,
        in_specs=[
            pl.BlockSpec((NC, bs, D), lambda i: (0, i, 0)),
            pl.BlockSpec((NC, bs, 1), lambda i: (0, i, 0)),
            pl.BlockSpec((D, D), lambda i: (0, 0)),
            pl.BlockSpec((1, D), lambda i: (0, 0)),
        ],
        out_specs=pl.BlockSpec((bs, D), lambda i: (i, 0)),
        out_shape=jax.ShapeDtypeStruct((N_NODES, D), jnp.float32),
    )(agg, deg.reshape(NC, N_TAB, 1), W, b.reshape(1, D))


def kernel(x, edge_index, W, b):
    e = edge_index.shape[1]
    extra = -e % NW
    if extra:
        ar = jnp.arange(extra, dtype=jnp.int32)
        padcols = jnp.stack(
            [ar % N_NODES, N_NODES + (ar % (N_TAB - N_NODES))])
        edge_index = jnp.concatenate([edge_index, padcols], axis=1)
        e += extra
    ept = e // NW
    agg, deg = _segment_mean_sc(x, edge_index, ept)
    return _linear_tc(agg, deg, W, b)


# speedup vs baseline: 1.0296x; 1.0296x over previous
import functools

import jax
import jax.numpy as jnp
from jax import lax
from jax.experimental import pallas as pl
from jax.experimental.pallas import tpu as pltpu
from jax.experimental.pallas import tpu_sc as plsc

N_NODES = 10000
D = 128

NC = 2
NS = 16
NW = NC * NS

K = 64
NBUF = 3
N_TAB = 10240
ROWS_PER_TILE = N_TAB // NS


def _sc_body(ept, e_tot, x_hbm, ei_hbm, agg_out, deg_out,
             sidx_all, didx_all, rows, ones_v, degz_v, gsems, agg_sh, deg_sh):
    n_full = ept // K
    rem = ept % K
    n_ring = (n_full // NBUF) * NBUF

    c = lax.axis_index("c")
    s = lax.axis_index("s")
    wid = c * NS + s
    base = wid * ept

    zero16 = jnp.zeros((16,), jnp.float32)
    one16 = jnp.ones((16,), jnp.float32)

    pltpu.sync_copy(ei_hbm.at[pl.ds(base, ept)], sidx_all)
    pltpu.sync_copy(ei_hbm.at[pl.ds(e_tot + base, ept)], didx_all)

    def _zrow(i, _):
        for j in range(D // 16):
            rows[0, i, pl.ds(j * 16, 16)] = zero16
        return 0
    lax.fori_loop(0, K, _zrow, 0)

    for j in range(K // 16):
        ones_v[pl.ds(j * 16, 16)] = one16

    def _zdeg(i, _):
        degz_v[pl.ds(i * 16, 16)] = zero16
        return 0
    lax.fori_loop(0, ROWS_PER_TILE // 16, _zdeg, 0)

    for bblk in range(ROWS_PER_TILE // K):
        pltpu.sync_copy(rows.at[0],
                        agg_sh.at[pl.ds(s * ROWS_PER_TILE + bblk * K, K)])
    pltpu.sync_copy(degz_v, deg_sh.at[pl.ds(s * ROWS_PER_TILE, ROWS_PER_TILE)])

    plsc.subcore_barrier()

    for b in range(min(NBUF, n_ring)):
        pltpu.async_copy(x_hbm.at[sidx_all.at[pl.ds(b * K, K)]], rows.at[b],
                         gsems[b])

    def _outer(go, _):
        for b in range(NBUF):
            g = go * NBUF + b
            pltpu.make_async_copy(x_hbm.at[sidx_all.at[pl.ds(g * K, K)]],
                                  rows.at[b], gsems[b]).wait()
            pltpu.sync_copy(rows.at[b],
                            agg_sh.at[didx_all.at[pl.ds(g * K, K)]], add=True)
            pltpu.sync_copy(ones_v,
                            deg_sh.at[didx_all.at[pl.ds(g * K, K)]], add=True)

            @pl.when(g + NBUF < n_ring)
            def _refill():
                pltpu.async_copy(
                    x_hbm.at[sidx_all.at[pl.ds((g + NBUF) * K, K)]],
                    rows.at[b], gsems[b])
        return 0
    lax.fori_loop(0, n_ring // NBUF, _outer, 0)

    for g in range(n_ring, n_full):
        b = g - n_ring
        pltpu.sync_copy(x_hbm.at[sidx_all.at[pl.ds(g * K, K)]], rows.at[b])
        pltpu.sync_copy(rows.at[b],
                        agg_sh.at[didx_all.at[pl.ds(g * K, K)]], add=True)
        pltpu.sync_copy(ones_v,
                        deg_sh.at[didx_all.at[pl.ds(g * K, K)]], add=True)
    if rem:
        off = n_full * K
        pltpu.sync_copy(x_hbm.at[sidx_all.at[pl.ds(off, rem)]],
                        rows.at[0, pl.ds(0, rem)])
        pltpu.sync_copy(rows.at[0, pl.ds(0, rem)],
                        agg_sh.at[didx_all.at[pl.ds(off, rem)]], add=True)
        pltpu.sync_copy(ones_v.at[pl.ds(0, rem)],
                        deg_sh.at[didx_all.at[pl.ds(off, rem)]], add=True)

    plsc.subcore_barrier()

    pltpu.sync_copy(agg_sh.at[pl.ds(s * ROWS_PER_TILE, ROWS_PER_TILE)],
                    agg_out.at[c, pl.ds(s * ROWS_PER_TILE, ROWS_PER_TILE)])
    pltpu.sync_copy(deg_sh.at[pl.ds(s * ROWS_PER_TILE, ROWS_PER_TILE)],
                    deg_out.at[c, pl.ds(s * ROWS_PER_TILE, ROWS_PER_TILE)])


def _segment_mean_sc(x, edge_index, ept):
    e_tot = edge_index.size // 2
    mesh = plsc.VectorSubcoreMesh(core_axis_name="c", subcore_axis_name="s")
    return pl.kernel(
        functools.partial(_sc_body, ept, e_tot),
        out_type=(
            jax.ShapeDtypeStruct((NC, N_TAB, D), jnp.float32),
            jax.ShapeDtypeStruct((NC, N_TAB), jnp.float32),
        ),
        mesh=mesh,
        scratch_types=[
            pltpu.VMEM((ept,), jnp.int32),
            pltpu.VMEM((ept,), jnp.int32),
            pltpu.VMEM((NBUF, K, D), jnp.float32),
            pltpu.VMEM((K,), jnp.float32),
            pltpu.VMEM((ROWS_PER_TILE,), jnp.float32),
            [pltpu.SemaphoreType.DMA] * NBUF,
            pltpu.VMEM_SHARED((N_TAB, D), jnp.float32),
            pltpu.VMEM_SHARED((N_TAB,), jnp.float32),
        ],
        name="gcn_segment_mean_sc",
    )(x, edge_index.reshape(-1))


def _tc_body(agg_ref, deg_ref, w_ref, b_ref, out_ref):
    agg = agg_ref[0] + agg_ref[1]
    deg = deg_ref[0] + deg_ref[1]
    mean = agg / jnp.maximum(deg, 1.0)[:, None]
    out_ref[...] = (
        jnp.dot(mean, w_ref[...].T, preferred_element_type=jnp.float32)
        + b_ref[...]
    )


def _linear_tc(agg, deg, W, b):
    bs = 1024
    grid = (N_TAB // bs,)
    return pl.pallas_call(
        _tc_body,
        grid=grid,
        in_specs=[
            pl.BlockSpec((NC, bs, D), lambda i: (0, i, 0)),
            pl.BlockSpec((NC, bs), lambda i: (0, i)),
            pl.BlockSpec((D, D), lambda i: (0, 0)),
            pl.BlockSpec((1, D), lambda i: (0, 0)),
        ],
        out_specs=pl.BlockSpec((bs, D), lambda i: (i, 0)),
        out_shape=jax.ShapeDtypeStruct((N_TAB, D), jnp.float32),
    )(agg, deg, W, b.reshape(1, D))


def kernel(x, edge_index, W, b):
    e = edge_index.shape[1]
    extra = -e % NW
    if extra:
        ar = jnp.arange(extra, dtype=jnp.int32)
        padcols = jnp.stack(
            [ar % N_NODES, N_NODES + (ar % (N_TAB - N_NODES))])
        edge_index = jnp.concatenate([edge_index, padcols], axis=1)
        e += extra
    ept = e // NW
    agg, deg = _segment_mean_sc(x, edge_index, ept)
    out = _linear_tc(agg, deg, W, b)
    return out[:N_NODES]
